# Initial kernel scaffold; baseline (speedup 1.0000x reference)
import jax
import jax.numpy as jnp
from jax.experimental import pallas as pl


def kernel(x, params):
    raise NotImplementedError("write your pallas kernel here")


# speedup vs baseline: 4.5371x; 4.5371x over previous
import functools
import math

import jax
import jax.numpy as jnp
from jax import lax
from jax.experimental import pallas as pl
from jax.experimental.pallas import tpu as pltpu
from jax.experimental.pallas import tpu_sc as plsc

_B = 4
_C = 96
_KNN = 9
_EPS = 1e-5

_STAGES = (
    (3136, 3136, 8, 392, 56),
    (784, 784, 2, 392, 56),
    (196, 208, 1, 208, 16),
    (49, 56, 1, 56, 8),
)


def _dot(a, b):
    return jnp.dot(a, b, preferred_element_type=jnp.float32)


def _conv1_body(xp_ref, w_ref, b_ref, y_ref, s1_ref, s2_ref):
    y = _dot(xp_ref[...], w_ref[...]) + b_ref[...]
    y_ref[...] = y
    p1 = jnp.sum(y, axis=0, keepdims=True)
    p2 = jnp.sum(y * y, axis=0, keepdims=True)

    @pl.when(pl.program_id(0) == 0)
    def _():
        s1_ref[...] = p1
        s2_ref[...] = p2

    @pl.when(pl.program_id(0) != 0)
    def _():
        s1_ref[...] += p1
        s2_ref[...] += p2


def _conv1_call(xp, w, b, block):
    rows, kdim = xp.shape
    cout = w.shape[1]
    grid = rows // block
    return pl.pallas_call(
        _conv1_body,
        grid=(grid,),
        in_specs=[
            pl.BlockSpec((block, kdim), lambda r: (r, 0)),
            pl.BlockSpec((kdim, cout), lambda r: (0, 0)),
            pl.BlockSpec((1, cout), lambda r: (0, 0)),
        ],
        out_specs=[
            pl.BlockSpec((block, cout), lambda r: (r, 0)),
            pl.BlockSpec((1, cout), lambda r: (0, 0)),
            pl.BlockSpec((1, cout), lambda r: (0, 0)),
        ],
        out_shape=[
            jax.ShapeDtypeStruct((rows, cout), jnp.float32),
            jax.ShapeDtypeStruct((1, cout), jnp.float32),
            jax.ShapeDtypeStruct((1, cout), jnp.float32),
        ],
    )(xp, w, b)


def _tapconv_body(xs_ref, w_ref, b_ref, y_ref, s1_ref, s2_ref):
    acc = _dot(xs_ref[0], w_ref[0]) + b_ref[...]
    for t in range(1, 9):
        acc = acc + _dot(xs_ref[t], w_ref[t])
    y_ref[...] = acc
    p1 = jnp.sum(acc, axis=0, keepdims=True)
    p2 = jnp.sum(acc * acc, axis=0, keepdims=True)

    @pl.when(pl.program_id(0) == 0)
    def _():
        s1_ref[...] = p1
        s2_ref[...] = p2

    @pl.when(pl.program_id(0) != 0)
    def _():
        s1_ref[...] += p1
        s2_ref[...] += p2


def _tapconv_call(xs, w, b, block):
    _, rows, cin = xs.shape
    cout = w.shape[2]
    grid = rows // block
    return pl.pallas_call(
        _tapconv_body,
        grid=(grid,),
        in_specs=[
            pl.BlockSpec((9, block, cin), lambda r: (0, r, 0)),
            pl.BlockSpec((9, cin, cout), lambda r: (0, 0, 0)),
            pl.BlockSpec((1, cout), lambda r: (0, 0)),
        ],
        out_specs=[
            pl.BlockSpec((block, cout), lambda r: (r, 0)),
            pl.BlockSpec((1, cout), lambda r: (0, 0)),
            pl.BlockSpec((1, cout), lambda r: (0, 0)),
        ],
        out_shape=[
            jax.ShapeDtypeStruct((rows, cout), jnp.float32),
            jax.ShapeDtypeStruct((1, cout), jnp.float32),
            jax.ShapeDtypeStruct((1, cout), jnp.float32),
        ],
    )(xs, w, b)


def _bn_terms(s1, s2, nrows, g, b):
    m = s1 / nrows
    v = s2 / nrows - m * m
    inv = g * lax.rsqrt(v + _EPS)
    return inv, b - m * inv


def _bnrelu_body(nrows, y_ref, s1_ref, s2_ref, g_ref, b_ref, o_ref):
    inv, shift = _bn_terms(s1_ref[...], s2_ref[...], nrows, g_ref[...], b_ref[...])
    o_ref[...] = jnp.maximum(y_ref[...] * inv + shift, 0.0)


def _bnrelu_call(y, s1, s2, g, b, block):
    rows, c = y.shape
    grid = rows // block
    return pl.pallas_call(
        functools.partial(_bnrelu_body, float(rows)),
        grid=(grid,),
        in_specs=[
            pl.BlockSpec((block, c), lambda r: (r, 0)),
            pl.BlockSpec((1, c), lambda r: (0, 0)),
            pl.BlockSpec((1, c), lambda r: (0, 0)),
            pl.BlockSpec((1, c), lambda r: (0, 0)),
            pl.BlockSpec((1, c), lambda r: (0, 0)),
        ],
        out_specs=pl.BlockSpec((block, c), lambda r: (r, 0)),
        out_shape=jax.ShapeDtypeStruct((rows, c), jnp.float32),
    )(y, s1, s2, g, b)


def _bnfc1_body(nrows, y_ref, s1_ref, s2_ref, g_ref, b_ref, add_ref,
                w_ref, wb_ref, sc_ref, o_ref):
    inv, shift = _bn_terms(s1_ref[...], s2_ref[...], nrows, g_ref[...], b_ref[...])
    xn = y_ref[...] * inv + shift + add_ref[...]
    sc_ref[...] = xn
    o_ref[...] = _dot(xn, w_ref[...]) + wb_ref[...]


def _bnfc1_call(y, s1, s2, g, b, add, w, wb, block):
    rows, c = y.shape
    grid = rows // block
    add_block = (block, c) if add.shape[0] == rows else (1, c)
    add_map = (lambda r: (r, 0)) if add.shape[0] == rows else (lambda r: (0, 0))
    return pl.pallas_call(
        functools.partial(_bnfc1_body, float(rows)),
        grid=(grid,),
        in_specs=[
            pl.BlockSpec((block, c), lambda r: (r, 0)),
            pl.BlockSpec((1, c), lambda r: (0, 0)),
            pl.BlockSpec((1, c), lambda r: (0, 0)),
            pl.BlockSpec((1, c), lambda r: (0, 0)),
            pl.BlockSpec((1, c), lambda r: (0, 0)),
            pl.BlockSpec(add_block, add_map),
            pl.BlockSpec((c, c), lambda r: (0, 0)),
            pl.BlockSpec((1, c), lambda r: (0, 0)),
        ],
        out_specs=[
            pl.BlockSpec((block, c), lambda r: (r, 0)),
            pl.BlockSpec((block, c), lambda r: (r, 0)),
        ],
        out_shape=[
            jax.ShapeDtypeStruct((rows, c), jnp.float32),
            jax.ShapeDtypeStruct((rows, c), jnp.float32),
        ],
    )(y, s1, s2, g, b, add, w, wb)


def _knn_body(n_log, n_pad, rb_n, n_rb, y_rows_ref, y_full_ref, idx_ref):
    b = pl.program_id(0)
    yf = y_full_ref[0]
    rows = y_rows_ref[0]
    x2f = jnp.sum(yf * yf, axis=1)
    x2r = jnp.sum(rows * rows, axis=1)
    d = (x2r[:, None] + x2f[None, :]
         - 2.0 * lax.dot_general(rows, yf, (((1,), (1,)), ((), ())),
                                 preferred_element_type=jnp.float32))
    col = lax.broadcasted_iota(jnp.int32, (rb_n, n_pad), 1)
    if n_log != n_pad:
        d = jnp.where(col < n_log, d, jnp.inf)
    base = b * n_pad
    big = jnp.int32(2 ** 30)
    for j in range(_KNN):
        m = jnp.min(d, axis=1, keepdims=True)
        idx_j = jnp.min(jnp.where(d == m, col, big), axis=1)
        idx_ref[0, j, :] = idx_j + base
        d = jnp.where(col == idx_j[:, None], jnp.inf, d)


def _knn_call(y_pad, n_log, n_pad, n_rb, rb_n):
    nblk = _B * n_rb
    return pl.pallas_call(
        functools.partial(_knn_body, n_log, n_pad, rb_n, n_rb),
        grid=(_B, n_rb),
        in_specs=[
            pl.BlockSpec((1, rb_n, _C), lambda b, r: (b, r, 0)),
            pl.BlockSpec((1, n_pad, _C), lambda b, r: (b, 0, 0)),
        ],
        out_specs=pl.BlockSpec((1, 16, rb_n), lambda b, r: (b * n_rb + r, 0, 0)),
        out_shape=jax.ShapeDtypeStruct((nblk, 16, rb_n), jnp.int32),
    )(y_pad, y_pad)


def _neighbor_max(y_flat, idx, n_pad, n_rb, rb_n, ch):
    t_sc = _B * n_pad
    cpb = rb_n // ch
    nchunks = t_sc // ch
    trips = -(-nchunks // 32)
    cvec = _C // 16
    mesh = plsc.VectorSubcoreMesh(core_axis_name="c", subcore_axis_name="s")
    idx_flat = idx.reshape(-1)

    @functools.partial(
        pl.kernel,
        mesh=mesh,
        out_type=jax.ShapeDtypeStruct((t_sc, 128), jnp.float32),
        scratch_types=(
            [pltpu.VMEM((9, ch), jnp.int32)]
            + [pltpu.VMEM((ch, 128), jnp.float32) for _ in range(9)]
            + [pltpu.VMEM((ch, 128), jnp.float32),
               pltpu.SemaphoreType.DMA, pltpu.SemaphoreType.DMA]
        ),
    )
    def gather_max(y_hbm, idx_hbm, out_hbm, idx_v, b0, b1, b2, b3, b4, b5,
                   b6, b7, b8, mx_v, isem, gsem):
        bufs = (b0, b1, b2, b3, b4, b5, b6, b7, b8)
        wid = lax.axis_index("s") * 2 + lax.axis_index("c")
        for t in range(trips):
            g = wid + 32 * t

            @pl.when(g < nchunks)
            def _():
                blk = g // cpb
                r0 = pl.multiple_of((g - blk * cpb) * ch, 8)
                icps = [pltpu.async_copy(
                    idx_hbm.at[pl.ds(blk * (16 * rb_n) + j * rb_n + r0, ch)],
                    idx_v.at[j], isem)
                    for j in range(9)]
                for cp in icps:
                    cp.wait()
                gcps = [pltpu.async_copy(
                    y_hbm.at[idx_v.at[j]], bufs[j], gsem) for j in range(9)]
                for cp in gcps:
                    cp.wait()

                def row_body(r, carry):
                    for c in range(cvec):
                        sl = pl.ds(c * 16, 16)
                        acc = bufs[0][r, sl]
                        for j in range(1, 9):
                            acc = jnp.maximum(acc, bufs[j][r, sl])
                        mx_v[r, sl] = acc
                    return carry

                lax.fori_loop(0, ch, row_body, 0)
                node0 = pl.multiple_of(blk * rb_n + r0, 8)
                pltpu.sync_copy(mx_v, out_hbm.at[pl.ds(node0, ch)])

    return gather_max(y_flat, idx_flat)


def _stage_tail_body(y_ref, mx_ref, sc_ref, g1m_ref, g2_ref, gb_ref,
                     f2w_ref, f2b_ref, fbw_ref, fbb_ref, faw_ref, fab_ref,
                     o_ref):
    h = jnp.maximum(_dot(y_ref[...], g1m_ref[...])
                    + _dot(mx_ref[...], g2_ref[...]) + gb_ref[...], 0.0)
    o = _dot(h, f2w_ref[...]) + f2b_ref[...] + sc_ref[...]
    f = jnp.maximum(_dot(o, fbw_ref[...]) + fbb_ref[...], 0.0)
    o_ref[...] = _dot(f, faw_ref[...]) + fab_ref[...] + o


def _stage_tail_call(y, mx, sc, g1m, g2, gb, f2w, f2b, fbw, fbb, faw, fab,
                     block):
    rows, c = y.shape
    hid = fbw.shape[1]
    grid = rows // block
    rmap = lambda r: (r, 0)
    wmap = lambda r: (0, 0)
    return pl.pallas_call(
        _stage_tail_body,
        grid=(grid,),
        in_specs=[
            pl.BlockSpec((block, c), rmap),
            pl.BlockSpec((block, c), rmap),
            pl.BlockSpec((block, c), rmap),
            pl.BlockSpec((c, c), wmap),
            pl.BlockSpec((c, c), wmap),
            pl.BlockSpec((1, c), wmap),
            pl.BlockSpec((c, c), wmap),
            pl.BlockSpec((1, c), wmap),
            pl.BlockSpec((c, hid), wmap),
            pl.BlockSpec((1, hid), wmap),
            pl.BlockSpec((hid, c), wmap),
            pl.BlockSpec((1, c), wmap),
        ],
        out_specs=pl.BlockSpec((block, c), rmap),
        out_shape=jax.ShapeDtypeStruct((rows, c), jnp.float32),
    )(y, mx, sc, g1m, g2, gb, f2w, f2b, fbw, fbb, faw, fab)


def _head_body(x_ref, w1_ref, b1_ref, w2_ref, b2_ref, o_ref):
    pooled = jnp.mean(x_ref[...], axis=1)
    h = jnp.maximum(_dot(pooled, w1_ref[...]) + b1_ref[...], 0.0)
    o_ref[...] = _dot(h, w2_ref[...]) + b2_ref[...]


def _head_call(x, w1, b1, w2, b2):
    b, n, c = x.shape
    h1 = w1.shape[1]
    h2 = w2.shape[1]
    return pl.pallas_call(
        _head_body,
        out_shape=jax.ShapeDtypeStruct((b, h2), jnp.float32),
    )(x, w1, b1, w2, b2)


def _tap_stack(img, out_hw):
    b, h, w, c = img.shape
    oh, ow = out_hw
    xp = jnp.pad(img, ((0, 0), (1, 1), (1, 1), (0, 0)))
    taps = []
    for ky in range(3):
        for kx in range(3):
            s = xp[:, ky:ky + 2 * oh:2, kx:kx + 2 * ow:2, :]
            taps.append(s.reshape(b * oh * ow, c))
    return jnp.stack(taps, axis=0)


def _tap_weights(w):
    return jnp.transpose(w, (2, 3, 1, 0)).reshape(9, w.shape[1], w.shape[0])


def kernel(x, params):
    st = params['stem']

    xt = jnp.transpose(x, (0, 2, 3, 1))
    xp = jnp.pad(xt, ((0, 0), (1, 1), (1, 1), (0, 0)))
    cols = [xp[:, ky:ky + 224:2, kx:kx + 224:2, :]
            for ky in range(3) for kx in range(3)]
    im2col = jnp.concatenate(cols, axis=-1).reshape(_B * 112 * 112, 27)
    w1 = jnp.transpose(st['c1_w'], (2, 3, 1, 0)).reshape(27, 48)
    y1, s1, s2 = _conv1_call(im2col, w1, st['c1_b'][None, :], 1568)
    a1 = _bnrelu_call(y1, s1, s2, st['bn1_g'][None, :], st['bn1_b'][None, :], 1568)

    xs2 = _tap_stack(a1.reshape(_B, 112, 112, 48), (56, 56))
    y2, s1, s2 = _tapconv_call(xs2, _tap_weights(st['c2_w']),
                               st['c2_b'][None, :], 1568)

    pos = jnp.transpose(params['pos'][0], (1, 2, 0)).reshape(1, 3136, _C)
    pos_t = jnp.broadcast_to(pos, (_B, 3136, _C)).reshape(_B * 3136, _C)

    sp = params['stages'][0]
    sc, y = _bnfc1_call(y2, s1, s2, st['bn2_g'][None, :], st['bn2_b'][None, :],
                        pos_t, sp['fc1_w'], sp['fc1_b'][None, :], 1568)

    hw = 56
    out = None
    for i in range(4):
        n_log, n_pad, n_rb, rb_n, ch = _STAGES[i]
        sp = params['stages'][i]
        if i > 0:
            d = params['downs'][i - 1]
            img = out.reshape(_B, hw, hw, _C)
            hw //= 2
            xs = _tap_stack(img, (hw, hw))
            yc, s1, s2 = _tapconv_call(xs, _tap_weights(d['w']),
                                       d['b'][None, :],
                                       min(784, _B * hw * hw))
            zeros = jnp.zeros((1, _C), jnp.float32)
            sc, y = _bnfc1_call(yc, s1, s2, d['g'][None, :], d['be'][None, :],
                                zeros, sp['fc1_w'], sp['fc1_b'][None, :],
                                min(784, _B * hw * hw))

        yb = y.reshape(_B, n_log, _C)
        if n_pad != n_log:
            yb = jnp.pad(yb, ((0, 0), (0, n_pad - n_log), (0, 0)))
        idx = _knn_call(yb, n_log, n_pad, n_rb, rb_n)
        y128 = jnp.pad(yb.reshape(_B * n_pad, _C), ((0, 0), (0, 128 - _C)))
        mx = _neighbor_max(y128, idx, n_pad, n_rb, rb_n, ch)
        mx = mx.reshape(_B, n_pad, 128)[:, :n_log, :_C].reshape(_B * n_log, _C)

        g1 = sp['g_w'][:_C, :]
        g2 = sp['g_w'][_C:, :]
        out = _stage_tail_call(
            y, mx, sc, g1 - g2, g2, sp['g_b'][None, :],
            sp['fc2_w'], sp['fc2_b'][None, :],
            sp['fb_w'], sp['fb_b'][None, :],
            sp['fa_w'], sp['fa_b'][None, :],
            min(784, _B * n_log * 1) if (_B * n_log) % 784 == 0 else _B * n_log)

    dp = params['dec']
    return _head_call(out.reshape(_B, 49, _C), dp['l1_w'], dp['l1_b'][None, :],
                      dp['l2_w'], dp['l2_b'][None, :])


# speedup vs baseline: 7.6561x; 1.6874x over previous
import functools
import math

import jax
import jax.numpy as jnp
from jax import lax
from jax.experimental import pallas as pl
from jax.experimental.pallas import tpu as pltpu
from jax.experimental.pallas import tpu_sc as plsc

_B = 4
_C = 96
_KNN = 9
_EPS = 1e-5

_STAGES = (
    (3136, 3136, 8, 392, 56),
    (784, 784, 2, 392, 56),
    (196, 208, 1, 208, 16),
    (49, 56, 1, 56, 8),
)


def _dot(a, b):
    return jnp.dot(a, b, preferred_element_type=jnp.float32)


def _conv1_body(xp_ref, w_ref, b_ref, y_ref, s1_ref, s2_ref):
    y = _dot(xp_ref[...], w_ref[...]) + b_ref[...]
    y_ref[...] = y
    p1 = jnp.sum(y, axis=0, keepdims=True)
    p2 = jnp.sum(y * y, axis=0, keepdims=True)

    @pl.when(pl.program_id(0) == 0)
    def _():
        s1_ref[...] = p1
        s2_ref[...] = p2

    @pl.when(pl.program_id(0) != 0)
    def _():
        s1_ref[...] += p1
        s2_ref[...] += p2


def _conv1_call(xp, w, b, block):
    rows, kdim = xp.shape
    cout = w.shape[1]
    grid = rows // block
    return pl.pallas_call(
        _conv1_body,
        grid=(grid,),
        in_specs=[
            pl.BlockSpec((block, kdim), lambda r: (r, 0)),
            pl.BlockSpec((kdim, cout), lambda r: (0, 0)),
            pl.BlockSpec((1, cout), lambda r: (0, 0)),
        ],
        out_specs=[
            pl.BlockSpec((block, cout), lambda r: (r, 0)),
            pl.BlockSpec((1, cout), lambda r: (0, 0)),
            pl.BlockSpec((1, cout), lambda r: (0, 0)),
        ],
        out_shape=[
            jax.ShapeDtypeStruct((rows, cout), jnp.float32),
            jax.ShapeDtypeStruct((1, cout), jnp.float32),
            jax.ShapeDtypeStruct((1, cout), jnp.float32),
        ],
    )(xp, w, b)


def _tapconv_body(xs_ref, w_ref, b_ref, y_ref, s1_ref, s2_ref):
    acc = _dot(xs_ref[0], w_ref[0]) + b_ref[...]
    for t in range(1, 9):
        acc = acc + _dot(xs_ref[t], w_ref[t])
    y_ref[...] = acc
    p1 = jnp.sum(acc, axis=0, keepdims=True)
    p2 = jnp.sum(acc * acc, axis=0, keepdims=True)

    @pl.when(pl.program_id(0) == 0)
    def _():
        s1_ref[...] = p1
        s2_ref[...] = p2

    @pl.when(pl.program_id(0) != 0)
    def _():
        s1_ref[...] += p1
        s2_ref[...] += p2


def _tapconv_call(xs, w, b, block):
    _, rows, cin = xs.shape
    cout = w.shape[2]
    grid = rows // block
    return pl.pallas_call(
        _tapconv_body,
        grid=(grid,),
        in_specs=[
            pl.BlockSpec((9, block, cin), lambda r: (0, r, 0)),
            pl.BlockSpec((9, cin, cout), lambda r: (0, 0, 0)),
            pl.BlockSpec((1, cout), lambda r: (0, 0)),
        ],
        out_specs=[
            pl.BlockSpec((block, cout), lambda r: (r, 0)),
            pl.BlockSpec((1, cout), lambda r: (0, 0)),
            pl.BlockSpec((1, cout), lambda r: (0, 0)),
        ],
        out_shape=[
            jax.ShapeDtypeStruct((rows, cout), jnp.float32),
            jax.ShapeDtypeStruct((1, cout), jnp.float32),
            jax.ShapeDtypeStruct((1, cout), jnp.float32),
        ],
    )(xs, w, b)


def _bn_terms(s1, s2, nrows, g, b):
    m = s1 / nrows
    v = s2 / nrows - m * m
    inv = g * lax.rsqrt(v + _EPS)
    return inv, b - m * inv


def _bnrelu_body(nrows, y_ref, s1_ref, s2_ref, g_ref, b_ref, o_ref):
    inv, shift = _bn_terms(s1_ref[...], s2_ref[...], nrows, g_ref[...], b_ref[...])
    o_ref[...] = jnp.maximum(y_ref[...] * inv + shift, 0.0)


def _bnrelu_call(y, s1, s2, g, b, block):
    rows, c = y.shape
    grid = rows // block
    return pl.pallas_call(
        functools.partial(_bnrelu_body, float(rows)),
        grid=(grid,),
        in_specs=[
            pl.BlockSpec((block, c), lambda r: (r, 0)),
            pl.BlockSpec((1, c), lambda r: (0, 0)),
            pl.BlockSpec((1, c), lambda r: (0, 0)),
            pl.BlockSpec((1, c), lambda r: (0, 0)),
            pl.BlockSpec((1, c), lambda r: (0, 0)),
        ],
        out_specs=pl.BlockSpec((block, c), lambda r: (r, 0)),
        out_shape=jax.ShapeDtypeStruct((rows, c), jnp.float32),
    )(y, s1, s2, g, b)


def _bnfc1_body(nrows, y_ref, s1_ref, s2_ref, g_ref, b_ref, add_ref,
                w_ref, wb_ref, sc_ref, o_ref):
    inv, shift = _bn_terms(s1_ref[...], s2_ref[...], nrows, g_ref[...], b_ref[...])
    xn = y_ref[...] * inv + shift + add_ref[...]
    sc_ref[...] = xn
    o_ref[...] = _dot(xn, w_ref[...]) + wb_ref[...]


def _bnfc1_call(y, s1, s2, g, b, add, w, wb, block):
    rows, c = y.shape
    grid = rows // block
    add_block = (block, c) if add.shape[0] == rows else (1, c)
    add_map = (lambda r: (r, 0)) if add.shape[0] == rows else (lambda r: (0, 0))
    return pl.pallas_call(
        functools.partial(_bnfc1_body, float(rows)),
        grid=(grid,),
        in_specs=[
            pl.BlockSpec((block, c), lambda r: (r, 0)),
            pl.BlockSpec((1, c), lambda r: (0, 0)),
            pl.BlockSpec((1, c), lambda r: (0, 0)),
            pl.BlockSpec((1, c), lambda r: (0, 0)),
            pl.BlockSpec((1, c), lambda r: (0, 0)),
            pl.BlockSpec(add_block, add_map),
            pl.BlockSpec((c, c), lambda r: (0, 0)),
            pl.BlockSpec((1, c), lambda r: (0, 0)),
        ],
        out_specs=[
            pl.BlockSpec((block, c), lambda r: (r, 0)),
            pl.BlockSpec((block, c), lambda r: (r, 0)),
        ],
        out_shape=[
            jax.ShapeDtypeStruct((rows, c), jnp.float32),
            jax.ShapeDtypeStruct((rows, c), jnp.float32),
        ],
    )(y, s1, s2, g, b, add, w, wb)


def _knn_body(n_log, n_pad, rb_n, n_rb, y_rows_ref, y_full_ref, idx_ref):
    b = pl.program_id(0)
    yf = y_full_ref[0]
    rows = y_rows_ref[0]
    x2f = jnp.sum(yf * yf, axis=1)
    x2r = jnp.sum(rows * rows, axis=1)
    d = (x2r[:, None] + x2f[None, :]
         - 2.0 * lax.dot_general(rows, yf, (((1,), (1,)), ((), ())),
                                 preferred_element_type=jnp.float32))
    colf = lax.broadcasted_iota(jnp.int32, (rb_n, n_pad), 1).astype(jnp.float32)
    if n_log != n_pad:
        d = jnp.where(colf < n_log, d, jnp.inf)
    base = b * n_pad
    bigf = jnp.float32(1e9)
    for j in range(_KNN):
        m = jnp.min(d, axis=1, keepdims=True)
        key = jnp.where(d == m, colf, bigf)
        idxf = jnp.min(key, axis=1)
        idx_ref[0, j, :] = idxf.astype(jnp.int32) + base
        d = jnp.where(key == idxf[:, None], jnp.inf, d)


def _knn_call(y_pad, n_log, n_pad, n_rb, rb_n):
    nblk = _B * n_rb
    return pl.pallas_call(
        functools.partial(_knn_body, n_log, n_pad, rb_n, n_rb),
        grid=(_B, n_rb),
        in_specs=[
            pl.BlockSpec((1, rb_n, _C), lambda b, r: (b, r, 0)),
            pl.BlockSpec((1, n_pad, _C), lambda b, r: (b, 0, 0)),
        ],
        out_specs=pl.BlockSpec((1, 16, rb_n), lambda b, r: (b * n_rb + r, 0, 0)),
        out_shape=jax.ShapeDtypeStruct((nblk, 16, rb_n), jnp.int32),
    )(y_pad, y_pad)


def _neighbor_max(y_flat, idx, n_pad, n_rb, rb_n, ch):
    t_sc = _B * n_pad
    cpb = rb_n // ch
    nchunks = t_sc // ch
    trips = -(-nchunks // 32)
    cvec = _C // 16
    mesh = plsc.VectorSubcoreMesh(core_axis_name="c", subcore_axis_name="s")
    idx_flat = idx.reshape(-1)

    @functools.partial(
        pl.kernel,
        mesh=mesh,
        out_type=jax.ShapeDtypeStruct((t_sc, 128), jnp.float32),
        scratch_types=(
            [pltpu.VMEM((9, ch), jnp.int32)]
            + [pltpu.VMEM((ch, 128), jnp.float32) for _ in range(9)]
            + [pltpu.VMEM((ch, 128), jnp.float32),
               pltpu.SemaphoreType.DMA, pltpu.SemaphoreType.DMA]
        ),
    )
    def gather_max(y_hbm, idx_hbm, out_hbm, idx_v, b0, b1, b2, b3, b4, b5,
                   b6, b7, b8, mx_v, isem, gsem):
        bufs = (b0, b1, b2, b3, b4, b5, b6, b7, b8)
        wid = lax.axis_index("s") * 2 + lax.axis_index("c")
        for t in range(trips):
            g = wid + 32 * t

            @pl.when(g < nchunks)
            def _():
                blk = g // cpb
                r0 = pl.multiple_of((g - blk * cpb) * ch, 8)
                icps = [pltpu.async_copy(
                    idx_hbm.at[pl.ds(blk * (16 * rb_n) + j * rb_n + r0, ch)],
                    idx_v.at[j], isem)
                    for j in range(9)]
                for cp in icps:
                    cp.wait()
                gcps = [pltpu.async_copy(
                    y_hbm.at[idx_v.at[j]], bufs[j], gsem) for j in range(9)]
                for cp in gcps:
                    cp.wait()

                def row_body(r, carry):
                    for c in range(cvec):
                        sl = pl.ds(c * 16, 16)
                        acc = bufs[0][r, sl]
                        for j in range(1, 9):
                            acc = jnp.maximum(acc, bufs[j][r, sl])
                        mx_v[r, sl] = acc
                    return carry

                lax.fori_loop(0, ch, row_body, 0)
                node0 = pl.multiple_of(blk * rb_n + r0, 8)
                pltpu.sync_copy(mx_v, out_hbm.at[pl.ds(node0, ch)])

    return gather_max(y_flat, idx_flat)


def _stage_tail_body(y_ref, mx_ref, sc_ref, g1m_ref, g2_ref, gb_ref,
                     f2w_ref, f2b_ref, fbw_ref, fbb_ref, faw_ref, fab_ref,
                     o_ref):
    h = jnp.maximum(_dot(y_ref[...], g1m_ref[...])
                    + _dot(mx_ref[...], g2_ref[...]) + gb_ref[...], 0.0)
    o = _dot(h, f2w_ref[...]) + f2b_ref[...] + sc_ref[...]
    f = jnp.maximum(_dot(o, fbw_ref[...]) + fbb_ref[...], 0.0)
    o_ref[...] = _dot(f, faw_ref[...]) + fab_ref[...] + o


def _stage_tail_call(y, mx, sc, g1m, g2, gb, f2w, f2b, fbw, fbb, faw, fab,
                     block):
    rows, c = y.shape
    hid = fbw.shape[1]
    grid = rows // block
    rmap = lambda r: (r, 0)
    wmap = lambda r: (0, 0)
    return pl.pallas_call(
        _stage_tail_body,
        grid=(grid,),
        in_specs=[
            pl.BlockSpec((block, c), rmap),
            pl.BlockSpec((block, c), rmap),
            pl.BlockSpec((block, c), rmap),
            pl.BlockSpec((c, c), wmap),
            pl.BlockSpec((c, c), wmap),
            pl.BlockSpec((1, c), wmap),
            pl.BlockSpec((c, c), wmap),
            pl.BlockSpec((1, c), wmap),
            pl.BlockSpec((c, hid), wmap),
            pl.BlockSpec((1, hid), wmap),
            pl.BlockSpec((hid, c), wmap),
            pl.BlockSpec((1, c), wmap),
        ],
        out_specs=pl.BlockSpec((block, c), rmap),
        out_shape=jax.ShapeDtypeStruct((rows, c), jnp.float32),
    )(y, mx, sc, g1m, g2, gb, f2w, f2b, fbw, fbb, faw, fab)


def _head_body(x_ref, w1_ref, b1_ref, w2_ref, b2_ref, o_ref):
    pooled = jnp.mean(x_ref[...], axis=1)
    h = jnp.maximum(_dot(pooled, w1_ref[...]) + b1_ref[...], 0.0)
    o_ref[...] = _dot(h, w2_ref[...]) + b2_ref[...]


def _head_call(x, w1, b1, w2, b2):
    b, n, c = x.shape
    h1 = w1.shape[1]
    h2 = w2.shape[1]
    return pl.pallas_call(
        _head_body,
        out_shape=jax.ShapeDtypeStruct((b, h2), jnp.float32),
    )(x, w1, b1, w2, b2)


def _taps(img, out_hw):
    b, h, w, c = img.shape
    oh, ow = out_hw
    xp = jnp.pad(img, ((0, 0), (1, 1), (1, 1), (0, 0)))
    ph, pw = (h + 2) // 2, (w + 2) // 2
    phases = jnp.transpose(xp.reshape(b, ph, 2, pw, 2, c), (2, 4, 0, 1, 3, 5))
    return [phases[ky % 2, kx % 2,
                   :, ky // 2:ky // 2 + oh, kx // 2:kx // 2 + ow, :]
            for ky in range(3) for kx in range(3)]


def _tap_stack(img, out_hw):
    b, _, _, c = img.shape
    oh, ow = out_hw
    return jnp.stack([t.reshape(b * oh * ow, c) for t in _taps(img, out_hw)],
                     axis=0)


def _tap_weights(w):
    return jnp.transpose(w, (2, 3, 1, 0)).reshape(9, w.shape[1], w.shape[0])


def kernel(x, params):
    st = params['stem']

    xt = jnp.transpose(x, (0, 2, 3, 1))
    im2col = jnp.concatenate(_taps(xt, (112, 112)),
                             axis=-1).reshape(_B * 112 * 112, 27)
    w1 = jnp.transpose(st['c1_w'], (2, 3, 1, 0)).reshape(27, 48)
    y1, s1, s2 = _conv1_call(im2col, w1, st['c1_b'][None, :], 1568)
    a1 = _bnrelu_call(y1, s1, s2, st['bn1_g'][None, :], st['bn1_b'][None, :], 1568)

    xs2 = _tap_stack(a1.reshape(_B, 112, 112, 48), (56, 56))
    y2, s1, s2 = _tapconv_call(xs2, _tap_weights(st['c2_w']),
                               st['c2_b'][None, :], 1568)

    pos = jnp.transpose(params['pos'][0], (1, 2, 0)).reshape(1, 3136, _C)
    pos_t = jnp.broadcast_to(pos, (_B, 3136, _C)).reshape(_B * 3136, _C)

    sp = params['stages'][0]
    sc, y = _bnfc1_call(y2, s1, s2, st['bn2_g'][None, :], st['bn2_b'][None, :],
                        pos_t, sp['fc1_w'], sp['fc1_b'][None, :], 1568)

    hw = 56
    out = None
    for i in range(4):
        n_log, n_pad, n_rb, rb_n, ch = _STAGES[i]
        sp = params['stages'][i]
        if i > 0:
            d = params['downs'][i - 1]
            img = out.reshape(_B, hw, hw, _C)
            hw //= 2
            xs = _tap_stack(img, (hw, hw))
            yc, s1, s2 = _tapconv_call(xs, _tap_weights(d['w']),
                                       d['b'][None, :],
                                       min(784, _B * hw * hw))
            zeros = jnp.zeros((1, _C), jnp.float32)
            sc, y = _bnfc1_call(yc, s1, s2, d['g'][None, :], d['be'][None, :],
                                zeros, sp['fc1_w'], sp['fc1_b'][None, :],
                                min(784, _B * hw * hw))

        yb = y.reshape(_B, n_log, _C)
        if n_pad != n_log:
            yb = jnp.pad(yb, ((0, 0), (0, n_pad - n_log), (0, 0)))
        idx = _knn_call(yb, n_log, n_pad, n_rb, rb_n)
        y128 = jnp.pad(yb.reshape(_B * n_pad, _C), ((0, 0), (0, 128 - _C)))
        mx = _neighbor_max(y128, idx, n_pad, n_rb, rb_n, ch)
        mx = mx.reshape(_B, n_pad, 128)[:, :n_log, :_C].reshape(_B * n_log, _C)

        g1 = sp['g_w'][:_C, :]
        g2 = sp['g_w'][_C:, :]
        out = _stage_tail_call(
            y, mx, sc, g1 - g2, g2, sp['g_b'][None, :],
            sp['fc2_w'], sp['fc2_b'][None, :],
            sp['fb_w'], sp['fb_b'][None, :],
            sp['fa_w'], sp['fa_b'][None, :],
            min(784, _B * n_log * 1) if (_B * n_log) % 784 == 0 else _B * n_log)

    dp = params['dec']
    return _head_call(out.reshape(_B, 49, _C), dp['l1_w'], dp['l1_b'][None, :],
                      dp['l2_w'], dp['l2_b'][None, :])
